# Initial kernel scaffold; baseline (speedup 1.0000x reference)
import functools

import jax
import jax.numpy as jnp
from jax import lax
from jax.experimental import pallas as pl
from jax.experimental.pallas import tpu as pltpu
from jax.experimental.pallas import tpu_sc as plsc

_PACK = 4
_MM_BLK = 2000


def _mm_body(x_ref, w_ref, o_ref):
    o_ref[...] = jnp.dot(x_ref[...], w_ref[...],
                         preferred_element_type=jnp.float32)


def _project_table(table, W):
    num, dim = table.shape
    packed_dim = _PACK * dim
    rows_packed = num // _PACK
    w_big = jnp.kron(jnp.eye(_PACK, dtype=W.dtype), W.T)
    packed = table.reshape(rows_packed, packed_dim)
    out = pl.pallas_call(
        _mm_body,
        grid=(rows_packed // _MM_BLK,),
        in_specs=[
            pl.BlockSpec((_MM_BLK, packed_dim), lambda i: (i, 0)),
            pl.BlockSpec((packed_dim, packed_dim), lambda i: (0, 0)),
        ],
        out_specs=pl.BlockSpec((_MM_BLK, packed_dim), lambda i: (i, 0)),
        out_shape=jax.ShapeDtypeStruct((rows_packed, packed_dim), jnp.float32),
    )(packed, w_big)
    return out.reshape(num, dim)


def _make_gather(n_flat, dim, chunk):
    info = plsc.get_sparse_core_info()
    nw = info.num_cores * info.num_subcores
    per_w = n_flat // nw
    n_chunks = per_w // chunk
    mesh = plsc.VectorSubcoreMesh(core_axis_name="c", subcore_axis_name="s")

    @functools.partial(
        pl.kernel,
        mesh=mesh,
        out_type=jax.ShapeDtypeStruct((n_flat, dim), jnp.float32),
        scratch_types=[
            pltpu.VMEM((chunk,), jnp.int32),
            pltpu.VMEM((chunk, dim), jnp.float32),
            pltpu.SemaphoreType.DMA,
        ],
    )
    def gather(tab_hbm, idx_hbm, out_hbm, idx_v, rows_v, sem):
        wid = lax.axis_index("s") * info.num_cores + lax.axis_index("c")
        base0 = wid * per_w

        def body(g, carry):
            base = base0 + g * chunk
            pltpu.sync_copy(idx_hbm.at[pl.ds(base, chunk)], idx_v)
            pltpu.async_copy(tab_hbm.at[idx_v], rows_v, sem).wait()
            pltpu.sync_copy(rows_v, out_hbm.at[pl.ds(base, chunk)])
            return carry

        lax.fori_loop(0, n_chunks, body, 0)

    return gather


def kernel(indexes, table, W):
    b, l = indexes.shape
    num, dim = table.shape
    P = _project_table(table, W)
    idx_flat = indexes.reshape(-1).astype(jnp.int32)
    out_flat = _make_gather(b * l, dim, 3200)(P, idx_flat)
    return out_flat.reshape(b, l, dim)


# speedup vs baseline: 11.6470x; 11.6470x over previous
import functools

import jax
import jax.numpy as jnp
from jax import lax
from jax.experimental import pallas as pl
from jax.experimental.pallas import tpu as pltpu
from jax.experimental.pallas import tpu_sc as plsc

_PACK = 4
_MM_BLK = 2000


def _mm_body(x_ref, w_ref, o_ref):
    o_ref[...] = jnp.dot(x_ref[...], w_ref[...],
                         preferred_element_type=jnp.float32)


def _project_table(table, W):
    num, dim = table.shape
    packed_dim = _PACK * dim
    rows_packed = num // _PACK
    w_big = jnp.kron(jnp.eye(_PACK, dtype=W.dtype), W.T)
    packed = table.reshape(rows_packed, packed_dim)
    out = pl.pallas_call(
        _mm_body,
        grid=(rows_packed // _MM_BLK,),
        in_specs=[
            pl.BlockSpec((_MM_BLK, packed_dim), lambda i: (i, 0)),
            pl.BlockSpec((packed_dim, packed_dim), lambda i: (0, 0)),
        ],
        out_specs=pl.BlockSpec((_MM_BLK, packed_dim), lambda i: (i, 0)),
        out_shape=jax.ShapeDtypeStruct((rows_packed, packed_dim), jnp.float32),
    )(packed, w_big)
    return out.reshape(num, dim)


def _make_gather(n_flat, dim, chunk):
    info = plsc.get_sparse_core_info()
    nw = info.num_cores * info.num_subcores
    per_w = n_flat // nw
    n_chunks = per_w // chunk
    mesh = plsc.VectorSubcoreMesh(core_axis_name="c", subcore_axis_name="s")

    @functools.partial(
        pl.kernel,
        mesh=mesh,
        out_type=jax.ShapeDtypeStruct((n_flat, dim), jnp.float32),
        scratch_types=[
            pltpu.VMEM((chunk,), jnp.int32),
            pltpu.VMEM((chunk, dim), jnp.float32),
            pltpu.SemaphoreType.DMA,
        ],
        compiler_params=pltpu.CompilerParams(use_tc_tiling_on_sc=False),
    )
    def gather(tab_hbm, idx_hbm, out_hbm, idx_v, rows_v, sem):
        wid = lax.axis_index("s") * info.num_cores + lax.axis_index("c")
        base0 = wid * per_w

        def body(g, carry):
            base = base0 + g * chunk
            pltpu.sync_copy(idx_hbm.at[pl.ds(base, chunk)], idx_v)
            pltpu.async_copy(tab_hbm.at[idx_v], rows_v, sem).wait()
            pltpu.sync_copy(rows_v, out_hbm.at[pl.ds(base, chunk)])
            return carry

        lax.fori_loop(0, n_chunks, body, 0)

    return gather


def kernel(indexes, table, W):
    b, l = indexes.shape
    num, dim = table.shape
    P = _project_table(table, W)
    idx_flat = indexes.reshape(-1).astype(jnp.int32)
    out_flat = _make_gather(b * l, dim, 3200)(P, idx_flat)
    return out_flat.reshape(b, l, dim)
